# baseline (speedup 1.0000x reference)
import functools
import math

import jax
import jax.numpy as jnp
from jax.experimental import pallas as pl
from jax.experimental.pallas import tpu as pltpu


def _bn_elu_body(n, co_p, full, z_ref, g_ref, be_ref, o_ref):
    z = z_ref[...]
    h = jnp.where(z > 0, z, jnp.exp(jnp.minimum(z, 0.0)) - 1.0)
    if full:
        hm = h
    else:
        r_i = jax.lax.broadcasted_iota(jnp.int32, z.shape, 0)
        l_i = jax.lax.broadcasted_iota(jnp.int32, z.shape, 1)
        lrow = (r_i * 128 + l_i) // co_p
        hm = jnp.where(lrow < n, h, 0.0)
    s = jnp.sum(hm, axis=0, keepdims=True)
    ss = jnp.sum(hm * hm, axis=0, keepdims=True)
    a_i = jax.lax.broadcasted_iota(jnp.int32, (128, 128), 0)
    b_i = jax.lax.broadcasted_iota(jnp.int32, (128, 128), 1)
    fold = ((a_i % co_p) == (b_i % co_p)).astype(jnp.float32)
    m = jax.lax.dot(s, fold, precision=jax.lax.Precision.HIGHEST) * (1.0 / n)
    sq = jax.lax.dot(ss, fold, precision=jax.lax.Precision.HIGHEST) * (1.0 / n)
    v = sq - m * m
    o_ref[...] = (h - m) * jax.lax.rsqrt(v + 1e-5) * g_ref[...] + be_ref[...]


def _bn_elu(z, g, be):
    n, co = z.shape
    co_p = co if co in (8, 16, 32, 64, 128) else 1 << max(3, (co - 1).bit_length())
    rep = 128 // co_p
    n_p = ((n + rep - 1) // rep) * rep
    zp = z
    if co_p != co:
        zp = jnp.pad(zp, ((0, 0), (0, co_p - co)))
    if n_p != n:
        zp = jnp.pad(zp, ((0, n_p - n), (0, 0)))
    z_rs = zp.reshape(n_p * co_p // 128, 128)
    gp = jnp.pad(g, (0, co_p - co)) if co_p != co else g
    bep = jnp.pad(be, (0, co_p - co)) if co_p != co else be
    g128 = jnp.tile(gp, rep).reshape(1, 128)
    be128 = jnp.tile(bep, rep).reshape(1, 128)
    out = pl.pallas_call(
        functools.partial(_bn_elu_body, n, co_p, n_p == n),
        out_shape=jax.ShapeDtypeStruct(z_rs.shape, jnp.float32),
    )(z_rs, g128, be128)
    out = out.reshape(n_p, co_p)
    return out[:n, :co]


def _gcn_conv(x, src, dst, ew, W, b, n):
    h = x @ W
    deg = jnp.zeros((n,), x.dtype).at[dst].add(ew) + 2.0
    dinv = jax.lax.rsqrt(deg)
    coef = dinv[src] * ew * dinv[dst]
    out = jnp.zeros_like(h).at[dst].add(h[src] * coef[:, None])
    out = out + h * (2.0 * dinv * dinv)[:, None]
    return out + b


def _topk_pool(x, src, dst, ew, p, ratio):
    n = x.shape[0]
    score = jnp.tanh(x @ p / jnp.linalg.norm(p))
    k = int(math.ceil(ratio * n))
    vals, perm = jax.lax.top_k(score, k)
    x2 = x[perm] * vals[:, None]
    mapping = jnp.full((n,), -1, dtype=src.dtype).at[perm].set(
        jnp.arange(k, dtype=src.dtype))
    ns = mapping[src]
    nd = mapping[dst]
    valid = (ns >= 0) & (nd >= 0)
    ew2 = ew * valid.astype(ew.dtype)
    ns = jnp.where(valid, ns, 0)
    nd = jnp.where(valid, nd, 0)
    return x2, ns, nd, ew2, k


def kernel(x, edge_index, params):
    src = edge_index[0]
    dst = edge_index[1]
    n = x.shape[0]
    ew = jnp.ones((src.shape[0],), x.dtype)
    pi = 0
    for i in range(8):
        z = _gcn_conv(x, src, dst, ew, params['W%d' % i], params['b%d' % i], n)
        x = _bn_elu(z, params['g%d' % i], params['be%d' % i])
        if i % 2 == 1:
            x, src, dst, ew, n = _topk_pool(
                x, src, dst, ew, params['p%d' % pi], 0.5)
            pi += 1
    z = _gcn_conv(x, src, dst, ew, params['W8'], params['b8'], n)
    return _bn_elu(z, params['g8'], params['be8'])
